# baseline (speedup 1.0000x reference)
import jax
import jax.numpy as jnp
from jax.experimental import pallas as pl

_BR = 1000


def _p1(a_ref, e0_ref, s1_ref, m1_ref):
    @pl.when(pl.program_id(0) == 0)
    def _():
        m1_ref[...] = jnp.zeros_like(m1_ref)

    a = a_ref[...]
    w = s1_ref[...] * e0_ref[...]
    m1_ref[...] += jax.lax.dot_general(
        a, w, (((0,), (0,)), ((), ())), preferred_element_type=jnp.float32)


def _p2(a_ref, m1_ref, e_ref, d_ref, s2_ref, e1_ref, m2_ref):
    @pl.when(pl.program_id(0) == 0)
    def _():
        m2_ref[...] = jnp.zeros_like(m2_ref)

    a = a_ref[...]
    y1 = e_ref[...] * m1_ref[...]
    t = jax.lax.dot_general(
        a, y1, (((1,), (0,)), ((), ())), preferred_element_type=jnp.float32)
    e1_ref[...] = d_ref[...] * t
    m2_ref[...] += jax.lax.dot_general(
        a, s2_ref[...] * t, (((0,), (0,)), ((), ())),
        preferred_element_type=jnp.float32)


def _p3(a_ref, m2_ref, e_ref, d_ref, e0_ref, e1_ref, out_ref):
    a = a_ref[...]
    y2 = e_ref[...] * m2_ref[...]
    t = jax.lax.dot_general(
        a, y2, (((1,), (0,)), ((), ())), preferred_element_type=jnp.float32)
    out_ref[...] = (e0_ref[...] + e1_ref[...] + d_ref[...] * t) * (1.0 / 3.0)


def kernel(users_embedding, product_embedding, adj_matrix, degreeV_matrix,
           degreeE_matrix, gate_user, gate_product):
    num_users, dim = users_embedding.shape
    n = num_users + product_embedding.shape[0]
    b = adj_matrix.shape[1]
    assert n % _BR == 0
    nsteps = n // _BR

    e0 = jnp.concatenate([users_embedding, product_embedding], axis=0)
    g = jnp.where(jnp.arange(n) < num_users, gate_user, gate_product)
    d = degreeV_matrix
    s1 = (d * g)[:, None].astype(jnp.float32)
    s2 = (d * d * g)[:, None].astype(jnp.float32)
    dcol = d[:, None]
    e = degreeE_matrix[:, None]

    m1 = pl.pallas_call(
        _p1,
        grid=(nsteps,),
        in_specs=[
            pl.BlockSpec((_BR, b), lambda k: (k, 0)),
            pl.BlockSpec((_BR, dim), lambda k: (k, 0)),
            pl.BlockSpec((_BR, 1), lambda k: (k, 0)),
        ],
        out_specs=pl.BlockSpec((b, dim), lambda k: (0, 0)),
        out_shape=jax.ShapeDtypeStruct((b, dim), jnp.float32),
    )(adj_matrix, e0, s1)

    e1, m2 = pl.pallas_call(
        _p2,
        grid=(nsteps,),
        in_specs=[
            pl.BlockSpec((_BR, b), lambda k: (k, 0)),
            pl.BlockSpec((b, dim), lambda k: (0, 0)),
            pl.BlockSpec((b, 1), lambda k: (0, 0)),
            pl.BlockSpec((_BR, 1), lambda k: (k, 0)),
            pl.BlockSpec((_BR, 1), lambda k: (k, 0)),
        ],
        out_specs=[
            pl.BlockSpec((_BR, dim), lambda k: (k, 0)),
            pl.BlockSpec((b, dim), lambda k: (0, 0)),
        ],
        out_shape=[
            jax.ShapeDtypeStruct((n, dim), jnp.float32),
            jax.ShapeDtypeStruct((b, dim), jnp.float32),
        ],
    )(adj_matrix, m1, e, dcol, s2)

    out = pl.pallas_call(
        _p3,
        grid=(nsteps,),
        in_specs=[
            pl.BlockSpec((_BR, b), lambda k: (k, 0)),
            pl.BlockSpec((b, dim), lambda k: (0, 0)),
            pl.BlockSpec((b, 1), lambda k: (0, 0)),
            pl.BlockSpec((_BR, 1), lambda k: (k, 0)),
            pl.BlockSpec((_BR, dim), lambda k: (k, 0)),
            pl.BlockSpec((_BR, dim), lambda k: (k, 0)),
        ],
        out_specs=pl.BlockSpec((_BR, dim), lambda k: (k, 0)),
        out_shape=jax.ShapeDtypeStruct((n, dim), jnp.float32),
    )(adj_matrix, m2, e, dcol, e0, e1)

    return (out[:num_users], out[num_users:])
